# Initial kernel scaffold; baseline (speedup 1.0000x reference)
import jax
import jax.numpy as jnp
from jax.experimental import pallas as pl


def kernel(hidden_states, ln1_w, q_w, q_b, k_w, k_b, v_w, v_b, o_w, o_b, sinks, ln2_w, router_w, router_b, gate_w, gate_b, up_w, up_b, down_w, down_b):
    raise NotImplementedError("write your pallas kernel here")


# speedup vs baseline: 1.4165x; 1.4165x over previous
import functools

import jax
import jax.numpy as jnp
import numpy as np
from jax import lax
from jax.experimental import pallas as pl
from jax.experimental.pallas import tpu as pltpu

EPS = 1e-06
THETA = 150000.0
ALPHA = 1.702
NEG = -1e30


def _rope_tables(S, HD):
    pos = jnp.arange(S, dtype=jnp.float32)
    inv = 1.0 / (THETA ** (jnp.arange(0, HD, 2, dtype=jnp.float32) / HD))
    fr = pos[:, None] * inv[None, :]
    cos = jnp.concatenate([jnp.cos(fr), jnp.cos(fr)], axis=-1)
    sin = jnp.concatenate([jnp.sin(fr), jnp.sin(fr)], axis=-1)
    return cos, sin


def _rot_half(x, HD):
    half = HD // 2
    lanes = lax.broadcasted_iota(jnp.int32, x.shape, 1) % HD
    lo = pltpu.roll(x, x.shape[1] - half, 1)
    hi = pltpu.roll(x, half, 1)
    return jnp.where(lanes < half, -lo, hi)


def _k1_body(x_ref, ln1_ref, w_ref, b_ref, cq_ref, sq_ref, ck_ref, sk_ref,
             q_out, k_out, v_out, *, DQ, DKV, HD):
    xb = x_ref[...]
    var = jnp.mean(xb * xb, axis=-1, keepdims=True)
    h = xb * lax.rsqrt(var + EPS) * ln1_ref[...]
    qkv = jnp.dot(h, w_ref[...],
                  preferred_element_type=jnp.float32)
    qkv = qkv + b_ref[...]
    q = qkv[:, :DQ]
    k = qkv[:, DQ:DQ + DKV]
    v = qkv[:, DQ + DKV:]
    q_out[...] = (q * cq_ref[...]
                  + _rot_half(q, HD) * sq_ref[...]).astype(jnp.bfloat16)
    k_out[...] = (k * ck_ref[...]
                  + _rot_half(k, HD) * sk_ref[...]).astype(jnp.bfloat16)
    v_out[...] = v.astype(jnp.bfloat16)


def _k2_body(q_ref, k_ref, v_ref, snk_ref, o_ref, *, BQ, S, scale):
    i = pl.program_id(1)
    q = q_ref[0]
    k = k_ref[0]
    s = lax.dot_general(q, k, (((1,), (1,)), ((), ())),
                        preferred_element_type=jnp.float32) * scale
    rows = lax.broadcasted_iota(jnp.int32, (BQ, S), 0) + i * BQ
    cols = lax.broadcasted_iota(jnp.int32, (BQ, S), 1)
    s = jnp.where(rows >= cols, s, NEG)
    snk = snk_ref[0, 0, 0]
    m = jnp.max(s, axis=-1, keepdims=True)
    m2 = jnp.maximum(m, snk)
    p = jnp.exp(s - m2)
    denom = jnp.sum(p, axis=-1, keepdims=True) + jnp.exp(snk - m2)
    probs = (p / denom).astype(jnp.bfloat16)
    o_ref[0] = jnp.dot(probs, v_ref[0],
                       preferred_element_type=jnp.float32).astype(jnp.bfloat16)


def _k3_body(att_ref, ow_ref, ob_ref, res_ref, ln2_ref, rw_ref, rb_ref,
             hid_out, x2_out, pw_out, *, E):
    a = att_ref[...]
    hid = res_ref[...] + jnp.dot(a, ow_ref[...],
                                 preferred_element_type=jnp.float32)
    hid = hid + ob_ref[...]
    hid_out[...] = hid
    var = jnp.mean(hid * hid, axis=-1, keepdims=True)
    x2 = hid * lax.rsqrt(var + EPS) * ln2_ref[...]
    x2_out[...] = x2
    rl = lax.dot_general(x2, rw_ref[...], (((1,), (0,)), ((), ())),
                         preferred_element_type=jnp.float32)
    rl = rl + rb_ref[...]
    lanes = lax.broadcasted_iota(jnp.int32, rl.shape, 1)
    m1 = jnp.max(rl, axis=-1, keepdims=True)
    a1 = jnp.min(jnp.where(rl == m1, lanes, E), axis=-1, keepdims=True)
    rl2 = jnp.where(lanes == a1, NEG, rl)
    m2 = jnp.max(rl2, axis=-1, keepdims=True)
    a2 = jnp.min(jnp.where(rl2 == m2, lanes, E), axis=-1, keepdims=True)
    e2 = jnp.exp(m2 - m1)
    p1 = 1.0 / (1.0 + e2)
    p2 = e2 * p1
    pw_out[...] = (jnp.where(lanes == a1, p1, 0.0)
                   + jnp.where(lanes == a2, p2, 0.0))


def _k4_body(x2_ref, hid_ref, pw_ref, gw_ref, gb_ref, uw_ref, ub_ref,
             dw_ref, db_ref, out_ref, acc_ref, *, BM, E):
    e = pl.program_id(0)
    i = pl.program_id(1)
    xb = x2_ref[pl.ds(i * BM, BM), :].astype(jnp.bfloat16)
    g = jnp.dot(xb, gw_ref[0], preferred_element_type=jnp.float32) + gb_ref[0]
    u = jnp.dot(xb, uw_ref[0], preferred_element_type=jnp.float32) + ub_ref[0]
    w0 = jnp.minimum(g, 7.0)
    w1 = jnp.clip(u, -7.0, 7.0)
    glu = w0 * (1.0 / (1.0 + jnp.exp(-ALPHA * w0)))
    inter = ((w1 + 1.0) * glu).astype(jnp.bfloat16)
    dn = jnp.dot(inter, dw_ref[0], preferred_element_type=jnp.float32)
    dn = dn + db_ref[0]
    lanes = lax.broadcasted_iota(jnp.int32, (BM, E), 1)
    pe = jnp.sum(jnp.where(lanes == e, pw_ref[pl.ds(i * BM, BM), :], 0.0),
                 axis=-1, keepdims=True)
    contrib = pe * dn

    @pl.when(e == 0)
    def _():
        acc_ref[i] = hid_ref[pl.ds(i * BM, BM), :] + contrib

    @pl.when(e > 0)
    def _():
        acc_ref[i] = acc_ref[i] + contrib

    @pl.when(e == E - 1)
    def _():
        out_ref[...] = acc_ref[i]


def kernel(hidden_states, ln1_w, q_w, q_b, k_w, k_b, v_w, v_b, o_w, o_b,
           sinks, ln2_w, router_w, router_b, gate_w, gate_b, up_w, up_b,
           down_w, down_b):
    B, S, D = hidden_states.shape
    H = sinks.shape[0]
    DQ = q_w.shape[1]
    DKV = k_w.shape[1]
    HD = DQ // H
    KV = DKV // HD
    E, _, F = gate_w.shape

    x = hidden_states.reshape(S, D)
    f32 = jnp.float32
    bf16 = jnp.bfloat16

    BS1 = min(512, S)
    qkv_w = jnp.concatenate([q_w, k_w, v_w], axis=1)
    qkv_b = jnp.concatenate([q_b, k_b, v_b]).reshape(1, -1)
    cos, sin = _rope_tables(S, HD)
    cq = jnp.tile(cos, (1, H))
    sq = jnp.tile(sin, (1, H))
    ck = jnp.tile(cos, (1, KV))
    sk = jnp.tile(sin, (1, KV))

    q, k, v = pl.pallas_call(
        functools.partial(_k1_body, DQ=DQ, DKV=DKV, HD=HD),
        grid=(S // BS1,),
        in_specs=[
            pl.BlockSpec((BS1, D), lambda i: (i, 0)),
            pl.BlockSpec((1, D), lambda i: (0, 0)),
            pl.BlockSpec((D, DQ + 2 * DKV), lambda i: (0, 0)),
            pl.BlockSpec((1, DQ + 2 * DKV), lambda i: (0, 0)),
            pl.BlockSpec((BS1, DQ), lambda i: (i, 0)),
            pl.BlockSpec((BS1, DQ), lambda i: (i, 0)),
            pl.BlockSpec((BS1, DKV), lambda i: (i, 0)),
            pl.BlockSpec((BS1, DKV), lambda i: (i, 0)),
        ],
        out_specs=[
            pl.BlockSpec((BS1, DQ), lambda i: (i, 0)),
            pl.BlockSpec((BS1, DKV), lambda i: (i, 0)),
            pl.BlockSpec((BS1, DKV), lambda i: (i, 0)),
        ],
        out_shape=[
            jax.ShapeDtypeStruct((S, DQ), bf16),
            jax.ShapeDtypeStruct((S, DKV), bf16),
            jax.ShapeDtypeStruct((S, DKV), bf16),
        ],
    )(x, ln1_w.reshape(1, D), qkv_w, qkv_b, cq, sq, ck, sk)

    qh = q.reshape(S, H, HD).transpose(1, 0, 2)
    kh = k.reshape(S, KV, HD).transpose(1, 0, 2)
    vh = v.reshape(S, KV, HD).transpose(1, 0, 2)
    snk3 = sinks.reshape(H, 1, 1)
    BQ = min(512, S)
    rep = H // KV
    attn = pl.pallas_call(
        functools.partial(_k2_body, BQ=BQ, S=S, scale=1.0 / float(np.sqrt(HD))),
        grid=(H, S // BQ),
        in_specs=[
            pl.BlockSpec((1, BQ, HD), lambda h, i: (h, i, 0)),
            pl.BlockSpec((1, S, HD), lambda h, i: (h // rep, 0, 0)),
            pl.BlockSpec((1, S, HD), lambda h, i: (h // rep, 0, 0)),
            pl.BlockSpec((1, 1, 1), lambda h, i: (h, 0, 0)),
        ],
        out_specs=pl.BlockSpec((1, BQ, HD), lambda h, i: (h, i, 0)),
        out_shape=jax.ShapeDtypeStruct((H, S, HD), bf16),
    )(qh, kh, vh, snk3)

    att2 = attn.transpose(1, 0, 2).reshape(S, DQ)

    BS3 = min(512, S)
    hid, x2, pw = pl.pallas_call(
        functools.partial(_k3_body, E=E),
        grid=(S // BS3,),
        in_specs=[
            pl.BlockSpec((BS3, DQ), lambda i: (i, 0)),
            pl.BlockSpec((DQ, D), lambda i: (0, 0)),
            pl.BlockSpec((1, D), lambda i: (0, 0)),
            pl.BlockSpec((BS3, D), lambda i: (i, 0)),
            pl.BlockSpec((1, D), lambda i: (0, 0)),
            pl.BlockSpec((D, E), lambda i: (0, 0)),
            pl.BlockSpec((1, E), lambda i: (0, 0)),
        ],
        out_specs=[
            pl.BlockSpec((BS3, D), lambda i: (i, 0)),
            pl.BlockSpec((BS3, D), lambda i: (i, 0)),
            pl.BlockSpec((BS3, E), lambda i: (i, 0)),
        ],
        out_shape=[
            jax.ShapeDtypeStruct((S, D), f32),
            jax.ShapeDtypeStruct((S, D), f32),
            jax.ShapeDtypeStruct((S, E), f32),
        ],
    )(att2, o_w, o_b.reshape(1, D), x, ln2_w.reshape(1, D),
      router_w, router_b.reshape(1, E))

    BM = min(512, S)
    NB = S // BM
    out = pl.pallas_call(
        functools.partial(_k4_body, BM=BM, E=E),
        grid=(E, NB),
        in_specs=[
            pl.BlockSpec((S, D), lambda e, i: (0, 0)),
            pl.BlockSpec((S, D), lambda e, i: (0, 0)),
            pl.BlockSpec((S, E), lambda e, i: (0, 0)),
            pl.BlockSpec((1, D, F), lambda e, i: (e, 0, 0)),
            pl.BlockSpec((1, 1, F), lambda e, i: (e, 0, 0)),
            pl.BlockSpec((1, D, F), lambda e, i: (e, 0, 0)),
            pl.BlockSpec((1, 1, F), lambda e, i: (e, 0, 0)),
            pl.BlockSpec((1, F, D), lambda e, i: (e, 0, 0)),
            pl.BlockSpec((1, 1, D), lambda e, i: (e, 0, 0)),
        ],
        out_specs=pl.BlockSpec((BM, D), lambda e, i: (i, 0)),
        out_shape=jax.ShapeDtypeStruct((S, D), f32),
        scratch_shapes=[pltpu.VMEM((NB, BM, D), f32)],
        compiler_params=pltpu.CompilerParams(
            vmem_limit_bytes=100 * 1024 * 1024),
    )(x2, hid, pw, gate_w.astype(bf16), gate_b.reshape(E, 1, F),
      up_w.astype(bf16), up_b.reshape(E, 1, F),
      down_w.astype(bf16), down_b.reshape(E, 1, D))

    return out.reshape(B, S, D)


# speedup vs baseline: 1.7153x; 1.2110x over previous
import functools

import jax
import jax.numpy as jnp
import numpy as np
from jax import lax
from jax.experimental import pallas as pl
from jax.experimental.pallas import tpu as pltpu
from jax.experimental.pallas import tpu_sc as plsc

EPS = 1e-06
THETA = 150000.0
ALPHA = 1.702
NEG = -1e30
NC, NS = 2, 16
NW = NC * NS


def _rope_tables(S, HD):
    pos = jnp.arange(S, dtype=jnp.float32)
    inv = 1.0 / (THETA ** (jnp.arange(0, HD, 2, dtype=jnp.float32) / HD))
    fr = pos[:, None] * inv[None, :]
    cos = jnp.concatenate([jnp.cos(fr), jnp.cos(fr)], axis=-1)
    sin = jnp.concatenate([jnp.sin(fr), jnp.sin(fr)], axis=-1)
    return cos, sin


def _rot_half(x, HD):
    half = HD // 2
    lanes = lax.broadcasted_iota(jnp.int32, x.shape, 1) % HD
    lo = pltpu.roll(x, x.shape[1] - half, 1)
    hi = pltpu.roll(x, half, 1)
    return jnp.where(lanes < half, -lo, hi)


def _k1_body(x_ref, ln1_ref, w_ref, b_ref, cq_ref, sq_ref, ck_ref, sk_ref,
             q_out, k_out, v_out, *, DQ, DKV, HD):
    xb = x_ref[...]
    var = jnp.mean(xb * xb, axis=-1, keepdims=True)
    h = xb * lax.rsqrt(var + EPS) * ln1_ref[...]
    qkv = jnp.dot(h, w_ref[...],
                  preferred_element_type=jnp.float32)
    qkv = qkv + b_ref[...]
    q = qkv[:, :DQ]
    k = qkv[:, DQ:DQ + DKV]
    v = qkv[:, DQ + DKV:]
    q_out[...] = (q * cq_ref[...]
                  + _rot_half(q, HD) * sq_ref[...]).astype(jnp.bfloat16)
    k_out[...] = (k * ck_ref[...]
                  + _rot_half(k, HD) * sk_ref[...]).astype(jnp.bfloat16)
    v_out[...] = v.astype(jnp.bfloat16)


def _k2_body(q_ref, k_ref, v_ref, snk_ref, o_ref, *, BQ, L, row0, scale):
    q = q_ref[0]
    k = k_ref[0]
    s = lax.dot_general(q, k, (((1,), (1,)), ((), ())),
                        preferred_element_type=jnp.float32) * scale
    rows = lax.broadcasted_iota(jnp.int32, (BQ, L), 0) + row0
    cols = lax.broadcasted_iota(jnp.int32, (BQ, L), 1)
    s = jnp.where(rows >= cols, s, NEG)
    snk = snk_ref[0, 0, 0]
    m = jnp.max(s, axis=-1, keepdims=True)
    m2 = jnp.maximum(m, snk)
    p = jnp.exp(s - m2)
    denom = jnp.sum(p, axis=-1, keepdims=True) + jnp.exp(snk - m2)
    probs = (p / denom).astype(jnp.bfloat16)
    o_ref[0] = jnp.dot(probs, v_ref[0],
                       preferred_element_type=jnp.float32).astype(jnp.bfloat16)


def _k3_body(att_ref, ow_ref, ob_ref, res_ref, ln2_ref, rw_ref, rb_ref,
             hid_out, x2_out, tk_out, *, E):
    a = att_ref[...]
    hid = res_ref[...] + jnp.dot(a, ow_ref[...],
                                 preferred_element_type=jnp.float32)
    hid = hid + ob_ref[...]
    hid_out[...] = hid
    var = jnp.mean(hid * hid, axis=-1, keepdims=True)
    x2 = hid * lax.rsqrt(var + EPS) * ln2_ref[...]
    x2_out[...] = x2
    rl = lax.dot_general(x2, rw_ref[...], (((1,), (0,)), ((), ())),
                         preferred_element_type=jnp.float32)
    rl = rl + rb_ref[...]
    lanes = lax.broadcasted_iota(jnp.int32, rl.shape, 1)
    m1 = jnp.max(rl, axis=-1, keepdims=True)
    a1 = jnp.min(jnp.where(rl == m1, lanes, E), axis=-1, keepdims=True)
    rl2 = jnp.where(lanes == a1, NEG, rl)
    m2 = jnp.max(rl2, axis=-1, keepdims=True)
    a2 = jnp.min(jnp.where(rl2 == m2, lanes, E), axis=-1, keepdims=True)
    e2 = jnp.exp(m2 - m1)
    p1 = 1.0 / (1.0 + e2)
    p2 = e2 * p1
    z = jnp.zeros_like(rl[:, :4])
    tk_out[...] = jnp.concatenate(
        [a1.astype(jnp.float32), a2.astype(jnp.float32), p1, p2, z], axis=1)


def _kr_body(tk_ref, L_ref, U_ref, pos_out, w_out, w0_out, w1_out,
             po_out, pc_out, *, S, E, BM):
    tkb = tk_ref[...]
    lanes = lax.broadcasted_iota(jnp.int32, (S, E), 1).astype(jnp.float32)
    e0 = tkb[:, 0:1]
    e1 = tkb[:, 1:2]
    oh0 = (lanes == e0).astype(jnp.bfloat16)
    oh1 = (lanes == e1).astype(jnp.bfloat16)
    cum0 = jnp.dot(L_ref[...], oh0, preferred_element_type=jnp.float32)
    cum1 = jnp.dot(L_ref[...], oh1, preferred_element_type=jnp.float32)
    tot0 = jnp.sum(oh0.astype(jnp.float32), axis=0, keepdims=True)
    tot1 = jnp.sum(oh1.astype(jnp.float32), axis=0, keepdims=True)
    c = tot0 + tot1
    pc = jnp.floor((c + (BM - 1)) / BM) * BM
    po = lax.dot_general(pc, U_ref[...], (((1,), (0,)), ((), ())),
                         precision=lax.Precision.HIGHEST,
                         preferred_element_type=jnp.float32)
    r0 = jnp.sum(jnp.where(lanes == e0, cum0 + po, 0.0), axis=1, keepdims=True)
    r1 = jnp.sum(jnp.where(lanes == e1, tot0 + cum1 + po, 0.0), axis=1,
                 keepdims=True)
    pos_out[...] = jnp.concatenate([r0, r1], axis=1).astype(jnp.int32)
    w_out[...] = tkb[:, 2:4]
    w0_out[...] = jnp.broadcast_to(tkb[:, 2:3], (S, 128))
    w1_out[...] = jnp.broadcast_to(tkb[:, 3:4], (S, 128))
    po_out[...] = po.astype(jnp.int32)
    pc_out[...] = pc.astype(jnp.int32)


def _ksg_body(x2_hbm, posf_hbm, w16_hbm, xg_hbm, wout_hbm, posv, rowsv,
              wrowv, sem, *, S, PAIRS_W, CH):
    wid = lax.axis_index("s") * NC + lax.axis_index("c")
    half = PAIRS_W * (NW // 2)
    base = wid * PAIRS_W
    tok_base = jnp.where(base >= half, base - half, base)
    for ch in range(PAIRS_W // CH):
        b = base + ch * CH
        tb = tok_base + ch * CH
        pltpu.sync_copy(posf_hbm.at[pl.ds(b, CH)], posv)
        pltpu.sync_copy(x2_hbm.at[pl.ds(tb, CH)], rowsv)
        pltpu.async_copy(rowsv, xg_hbm.at[posv], sem).wait()
        pltpu.sync_copy(w16_hbm.at[pl.ds(b, CH)], wrowv)
        pltpu.async_copy(wrowv, wout_hbm.at[posv], sem).wait()


def _kc_body(pout_hbm, posf_hbm, pout2_hbm, posv, rowsv, sem, *,
             PAIRS_W, CH):
    wid = lax.axis_index("s") * NC + lax.axis_index("c")
    base = wid * PAIRS_W
    for ch in range(PAIRS_W // CH):
        b = base + ch * CH
        pltpu.sync_copy(posf_hbm.at[pl.ds(b, CH)], posv)
        pltpu.async_copy(pout_hbm.at[posv], rowsv, sem).wait()
        pltpu.sync_copy(rowsv, pout2_hbm.at[pl.ds(b, CH)])


def _ka_body(hid_ref, p0_ref, p1_ref, out_ref):
    out_ref[...] = hid_ref[...] + p0_ref[...] + p1_ref[...]


def _kf_body(po_ref, pc_ref, xg_ref, wv_ref, gw_ref, gb_ref, uw_ref, ub_ref,
             dw_ref, db_ref, out_ref, *, BM, E):
    g = pl.program_id(0)
    nb = (po_ref[E - 1] + pc_ref[E - 1]) // BM

    @pl.when(g < nb)
    def _():
        xb = xg_ref[...]
        gg = jnp.dot(xb, gw_ref[0], preferred_element_type=jnp.float32)
        gg = gg + gb_ref[0]
        u = jnp.dot(xb, uw_ref[0], preferred_element_type=jnp.float32)
        u = u + ub_ref[0]
        w0 = jnp.minimum(gg, 7.0)
        w1 = jnp.clip(u, -7.0, 7.0)
        glu = w0 * (1.0 / (1.0 + jnp.exp(-ALPHA * w0)))
        inter = (w1 + 1.0) * glu
        dn = jnp.dot(inter, dw_ref[0], preferred_element_type=jnp.float32)
        dn = dn + db_ref[0]
        out_ref[...] = dn * wv_ref[:, 0:1]


def _run_sc_scatter(x2, posf, w16, S, D, PAD):
    PAIRS_W = (2 * S) // NW
    CH = 64
    while PAIRS_W % CH:
        CH //= 2
    mesh = plsc.VectorSubcoreMesh(core_axis_name="c", subcore_axis_name="s")
    f = functools.partial(
        pl.kernel,
        functools.partial(_ksg_body, S=S, PAIRS_W=PAIRS_W, CH=CH),
        out_type=[jax.ShapeDtypeStruct((PAD, D), jnp.float32),
                  jax.ShapeDtypeStruct((PAD, 128), jnp.float32)],
        mesh=mesh,
        scratch_types=[pltpu.VMEM((CH,), jnp.int32),
                       pltpu.VMEM((CH, D), jnp.float32),
                       pltpu.VMEM((CH, 128), jnp.float32),
                       pltpu.SemaphoreType.DMA],
    )()
    return f(x2, posf, w16)


def _run_sc_combine(hid, pairout, posf, S, D):
    PAIRS_W = (2 * S) // NW
    CH = 64
    while PAIRS_W % CH:
        CH //= 2
    mesh = plsc.VectorSubcoreMesh(core_axis_name="c", subcore_axis_name="s")
    f = functools.partial(
        pl.kernel,
        functools.partial(_kc_body, PAIRS_W=PAIRS_W, CH=CH),
        out_type=jax.ShapeDtypeStruct((2 * S, D), jnp.float32),
        mesh=mesh,
        scratch_types=[pltpu.VMEM((CH,), jnp.int32),
                       pltpu.VMEM((CH, D), jnp.float32),
                       pltpu.SemaphoreType.DMA],
    )()
    pout2 = f(pairout, posf)
    BS = min(512, S)
    NI = S // BS
    return pl.pallas_call(
        _ka_body,
        grid=(NI,),
        in_specs=[
            pl.BlockSpec((BS, D), lambda i: (i, 0)),
            pl.BlockSpec((BS, D), lambda i: (i, 0)),
            pl.BlockSpec((BS, D), lambda i, NI=NI: (i + NI, 0)),
        ],
        out_specs=pl.BlockSpec((BS, D), lambda i: (i, 0)),
        out_shape=jax.ShapeDtypeStruct((S, D), jnp.float32),
    )(hid, pout2, pout2)


def kernel(hidden_states, ln1_w, q_w, q_b, k_w, k_b, v_w, v_b, o_w, o_b,
           sinks, ln2_w, router_w, router_b, gate_w, gate_b, up_w, up_b,
           down_w, down_b):
    B, S, D = hidden_states.shape
    H = sinks.shape[0]
    DQ = q_w.shape[1]
    DKV = k_w.shape[1]
    HD = DQ // H
    KV = DKV // HD
    E, _, F = gate_w.shape
    BM = 256
    PAD = 2 * S + E * BM
    G = PAD // BM

    x = hidden_states.reshape(S, D)
    f32 = jnp.float32
    bf16 = jnp.bfloat16

    BS1 = min(512, S)
    qkv_w = jnp.concatenate([q_w, k_w, v_w], axis=1)
    qkv_b = jnp.concatenate([q_b, k_b, v_b]).reshape(1, -1)
    cos, sin = _rope_tables(S, HD)
    cq = jnp.tile(cos, (1, H))
    sq = jnp.tile(sin, (1, H))
    ck = jnp.tile(cos, (1, KV))
    sk = jnp.tile(sin, (1, KV))

    q, k, v = pl.pallas_call(
        functools.partial(_k1_body, DQ=DQ, DKV=DKV, HD=HD),
        grid=(S // BS1,),
        in_specs=[
            pl.BlockSpec((BS1, D), lambda i: (i, 0)),
            pl.BlockSpec((1, D), lambda i: (0, 0)),
            pl.BlockSpec((D, DQ + 2 * DKV), lambda i: (0, 0)),
            pl.BlockSpec((1, DQ + 2 * DKV), lambda i: (0, 0)),
            pl.BlockSpec((BS1, DQ), lambda i: (i, 0)),
            pl.BlockSpec((BS1, DQ), lambda i: (i, 0)),
            pl.BlockSpec((BS1, DKV), lambda i: (i, 0)),
            pl.BlockSpec((BS1, DKV), lambda i: (i, 0)),
        ],
        out_specs=[
            pl.BlockSpec((BS1, DQ), lambda i: (i, 0)),
            pl.BlockSpec((BS1, DKV), lambda i: (i, 0)),
            pl.BlockSpec((BS1, DKV), lambda i: (i, 0)),
        ],
        out_shape=[
            jax.ShapeDtypeStruct((S, DQ), bf16),
            jax.ShapeDtypeStruct((S, DKV), bf16),
            jax.ShapeDtypeStruct((S, DKV), bf16),
        ],
    )(x, ln1_w.reshape(1, D), qkv_w, qkv_b, cq, sq, ck, sk)

    qh = q.reshape(S, H, HD).transpose(1, 0, 2)
    kh = k.reshape(S, KV, HD).transpose(1, 0, 2)
    vh = v.reshape(S, KV, HD).transpose(1, 0, 2)
    snk3 = sinks.reshape(H, 1, 1)
    BQ = min(512, S)
    rep = H // KV
    scale = 1.0 / float(np.sqrt(HD))
    parts = []
    for blk in range(S // BQ):
        L = (blk + 1) * BQ
        part = pl.pallas_call(
            functools.partial(_k2_body, BQ=BQ, L=L, row0=blk * BQ,
                              scale=scale),
            grid=(H,),
            in_specs=[
                pl.BlockSpec((1, BQ, HD), lambda h: (h, 0, 0)),
                pl.BlockSpec((1, L, HD), lambda h: (h // rep, 0, 0)),
                pl.BlockSpec((1, L, HD), lambda h: (h // rep, 0, 0)),
                pl.BlockSpec((1, 1, 1), lambda h: (h, 0, 0)),
            ],
            out_specs=pl.BlockSpec((1, BQ, HD), lambda h: (h, 0, 0)),
            out_shape=jax.ShapeDtypeStruct((H, BQ, HD), bf16),
        )(lax.slice_in_dim(qh, blk * BQ, L, axis=1),
          lax.slice_in_dim(kh, 0, L, axis=1),
          lax.slice_in_dim(vh, 0, L, axis=1), snk3)
        parts.append(part)
    attn = jnp.concatenate(parts, axis=1)

    att2 = attn.transpose(1, 0, 2).reshape(S, DQ)

    BS3 = min(512, S)
    hid, x2, tk = pl.pallas_call(
        functools.partial(_k3_body, E=E),
        grid=(S // BS3,),
        in_specs=[
            pl.BlockSpec((BS3, DQ), lambda i: (i, 0)),
            pl.BlockSpec((DQ, D), lambda i: (0, 0)),
            pl.BlockSpec((1, D), lambda i: (0, 0)),
            pl.BlockSpec((BS3, D), lambda i: (i, 0)),
            pl.BlockSpec((1, D), lambda i: (0, 0)),
            pl.BlockSpec((D, E), lambda i: (0, 0)),
            pl.BlockSpec((1, E), lambda i: (0, 0)),
        ],
        out_specs=[
            pl.BlockSpec((BS3, D), lambda i: (i, 0)),
            pl.BlockSpec((BS3, D), lambda i: (i, 0)),
            pl.BlockSpec((BS3, E), lambda i: (i, 0)),
        ],
        out_shape=[
            jax.ShapeDtypeStruct((S, D), f32),
            jax.ShapeDtypeStruct((S, D), f32),
            jax.ShapeDtypeStruct((S, E), f32),
        ],
    )(att2, o_w, o_b.reshape(1, D), x, ln2_w.reshape(1, D),
      router_w, router_b.reshape(1, E))

    Lnp = np.tril(np.ones((S, S), np.float32), -1).astype(jnp.bfloat16)
    Unp = np.triu(np.ones((E, E), np.float32), 1)
    pos, w2, w0b, w1b, po, pc = pl.pallas_call(
        functools.partial(_kr_body, S=S, E=E, BM=BM),
        grid=(1,),
        in_specs=[
            pl.BlockSpec((S, E), lambda i: (0, 0)),
            pl.BlockSpec((S, S), lambda i: (0, 0)),
            pl.BlockSpec((E, E), lambda i: (0, 0)),
        ],
        out_specs=[
            pl.BlockSpec((S, 2), lambda i: (0, 0)),
            pl.BlockSpec((S, 2), lambda i: (0, 0)),
            pl.BlockSpec((S, 128), lambda i: (0, 0)),
            pl.BlockSpec((S, 128), lambda i: (0, 0)),
            pl.BlockSpec((1, E), lambda i: (0, 0)),
            pl.BlockSpec((1, E), lambda i: (0, 0)),
        ],
        out_shape=[
            jax.ShapeDtypeStruct((S, 2), jnp.int32),
            jax.ShapeDtypeStruct((S, 2), f32),
            jax.ShapeDtypeStruct((S, 128), f32),
            jax.ShapeDtypeStruct((S, 128), f32),
            jax.ShapeDtypeStruct((1, E), jnp.int32),
            jax.ShapeDtypeStruct((1, E), jnp.int32),
        ],
    )(tk, Lnp, Unp)

    posf = pos.T.reshape(2 * S)

    w16 = jnp.concatenate([w0b, w1b], axis=0)
    xg, wv16 = _run_sc_scatter(x2, posf, w16, S, D, PAD)

    grid_spec = pltpu.PrefetchScalarGridSpec(
        num_scalar_prefetch=2,
        grid=(G,),
        in_specs=[
            pl.BlockSpec((BM, D), lambda g, po_r, pc_r: (g, 0)),
            pl.BlockSpec((BM, 128), lambda g, po_r, pc_r: (g, 0)),
            pl.BlockSpec((1, D, F),
                         lambda g, po_r, pc_r: (_e_of(g, po_r, BM, E), 0, 0)),
            pl.BlockSpec((1, 1, F),
                         lambda g, po_r, pc_r: (_e_of(g, po_r, BM, E), 0, 0)),
            pl.BlockSpec((1, D, F),
                         lambda g, po_r, pc_r: (_e_of(g, po_r, BM, E), 0, 0)),
            pl.BlockSpec((1, 1, F),
                         lambda g, po_r, pc_r: (_e_of(g, po_r, BM, E), 0, 0)),
            pl.BlockSpec((1, F, D),
                         lambda g, po_r, pc_r: (_e_of(g, po_r, BM, E), 0, 0)),
            pl.BlockSpec((1, 1, D),
                         lambda g, po_r, pc_r: (_e_of(g, po_r, BM, E), 0, 0)),
        ],
        out_specs=pl.BlockSpec((BM, D), lambda g, po_r, pc_r: (g, 0)),
    )
    pairout = pl.pallas_call(
        functools.partial(_kf_body, BM=BM, E=E),
        grid_spec=grid_spec,
        out_shape=jax.ShapeDtypeStruct((PAD, D), f32),
        compiler_params=pltpu.CompilerParams(
            vmem_limit_bytes=100 * 1024 * 1024),
    )(po.reshape(E), pc.reshape(E), xg, wv16,
      gate_w, gate_b.reshape(E, 1, F),
      up_w, up_b.reshape(E, 1, F),
      down_w, down_b.reshape(E, 1, D))

    out = _run_sc_combine(hid, pairout, posf, S, D)
    return out.reshape(B, S, D)


def _e_of(g, po_ref, BM, E):
    acc = jnp.int32(-1)
    for ee in range(E):
        acc = acc + jnp.where(po_ref[ee] <= g * BM, 1, 0).astype(jnp.int32)
    return acc


# speedup vs baseline: 1.7577x; 1.0247x over previous
import functools

import jax
import jax.numpy as jnp
import numpy as np
from jax import lax
from jax.experimental import pallas as pl
from jax.experimental.pallas import tpu as pltpu
from jax.experimental.pallas import tpu_sc as plsc

EPS = 1e-06
THETA = 150000.0
ALPHA = 1.702
NEG = -1e30
NC, NS = 2, 16
NW = NC * NS


def _rope_tables(S, HD):
    pos = jnp.arange(S, dtype=jnp.float32)
    inv = 1.0 / (THETA ** (jnp.arange(0, HD, 2, dtype=jnp.float32) / HD))
    fr = pos[:, None] * inv[None, :]
    cos = jnp.concatenate([jnp.cos(fr), jnp.cos(fr)], axis=-1)
    sin = jnp.concatenate([jnp.sin(fr), jnp.sin(fr)], axis=-1)
    return cos, sin


def _rot_half(x, HD):
    half = HD // 2
    lanes = lax.broadcasted_iota(jnp.int32, x.shape, 1) % HD
    lo = pltpu.roll(x, x.shape[1] - half, 1)
    hi = pltpu.roll(x, half, 1)
    return jnp.where(lanes < half, -lo, hi)


def _k1_body(x_ref, ln1_ref, w_ref, b_ref, cq_ref, sq_ref, ck_ref, sk_ref,
             q_out, k_out, v_out, *, DQ, DKV, HD):
    xb = x_ref[...]
    var = jnp.mean(xb * xb, axis=-1, keepdims=True)
    h = xb * lax.rsqrt(var + EPS) * ln1_ref[...]
    qkv = jnp.dot(h, w_ref[...],
                  preferred_element_type=jnp.float32)
    qkv = qkv + b_ref[...]
    q = qkv[:, :DQ]
    k = qkv[:, DQ:DQ + DKV]
    v = qkv[:, DQ + DKV:]
    q_out[...] = (q * cq_ref[...]
                  + _rot_half(q, HD) * sq_ref[...]).astype(jnp.bfloat16)
    k_out[...] = (k * ck_ref[...]
                  + _rot_half(k, HD) * sk_ref[...]).astype(jnp.bfloat16)
    v_out[...] = v.astype(jnp.bfloat16)


def _k2_body(q_ref, k_ref, v_ref, snk_ref, o_ref, *, BQ, L, row0, scale):
    q = q_ref[0]
    k = k_ref[0]
    s = lax.dot_general(q, k, (((1,), (1,)), ((), ())),
                        preferred_element_type=jnp.float32) * scale
    rows = lax.broadcasted_iota(jnp.int32, (BQ, L), 0) + row0
    cols = lax.broadcasted_iota(jnp.int32, (BQ, L), 1)
    s = jnp.where(rows >= cols, s, NEG)
    snk = snk_ref[0, 0, 0]
    m = jnp.max(s, axis=-1, keepdims=True)
    m2 = jnp.maximum(m, snk)
    p = jnp.exp(s - m2)
    denom = jnp.sum(p, axis=-1, keepdims=True) + jnp.exp(snk - m2)
    probs = (p / denom).astype(jnp.bfloat16)
    o_ref[0] = jnp.dot(probs, v_ref[0],
                       preferred_element_type=jnp.float32).astype(jnp.bfloat16)


def _k3_body(att_ref, ow_ref, ob_ref, res_ref, ln2_ref, rw_ref, rb_ref,
             hid_out, x2_out, tk_out, *, E):
    a = att_ref[...]
    hid = res_ref[...] + jnp.dot(a, ow_ref[...],
                                 preferred_element_type=jnp.float32)
    hid = hid + ob_ref[...]
    hid_out[...] = hid
    var = jnp.mean(hid * hid, axis=-1, keepdims=True)
    x2 = hid * lax.rsqrt(var + EPS) * ln2_ref[...]
    x2_out[...] = x2
    rl = lax.dot_general(x2, rw_ref[...], (((1,), (0,)), ((), ())),
                         preferred_element_type=jnp.float32)
    rl = rl + rb_ref[...]
    lanes = lax.broadcasted_iota(jnp.int32, rl.shape, 1)
    m1 = jnp.max(rl, axis=-1, keepdims=True)
    a1 = jnp.min(jnp.where(rl == m1, lanes, E), axis=-1, keepdims=True)
    rl2 = jnp.where(lanes == a1, NEG, rl)
    m2 = jnp.max(rl2, axis=-1, keepdims=True)
    a2 = jnp.min(jnp.where(rl2 == m2, lanes, E), axis=-1, keepdims=True)
    e2 = jnp.exp(m2 - m1)
    p1 = 1.0 / (1.0 + e2)
    p2 = e2 * p1
    z = jnp.zeros_like(rl[:, :4])
    tk_out[...] = jnp.concatenate(
        [a1.astype(jnp.float32), a2.astype(jnp.float32), p1, p2, z], axis=1)


def _kr_body(tk_ref, L_ref, U_ref, pos_out, w_out, w0_out, w1_out,
             po_out, pc_out, *, S, E, BM):
    tkb = tk_ref[...]
    lanes = lax.broadcasted_iota(jnp.int32, (S, E), 1).astype(jnp.float32)
    e0 = tkb[:, 0:1]
    e1 = tkb[:, 1:2]
    oh0 = (lanes == e0).astype(jnp.bfloat16)
    oh1 = (lanes == e1).astype(jnp.bfloat16)
    cum0 = jnp.dot(L_ref[...], oh0, preferred_element_type=jnp.float32)
    cum1 = jnp.dot(L_ref[...], oh1, preferred_element_type=jnp.float32)
    tot0 = jnp.sum(oh0.astype(jnp.float32), axis=0, keepdims=True)
    tot1 = jnp.sum(oh1.astype(jnp.float32), axis=0, keepdims=True)
    c = tot0 + tot1
    pc = jnp.floor((c + (BM - 1)) / BM) * BM
    po = lax.dot_general(pc, U_ref[...], (((1,), (0,)), ((), ())),
                         precision=lax.Precision.HIGHEST,
                         preferred_element_type=jnp.float32)
    r0 = jnp.sum(jnp.where(lanes == e0, cum0 + po, 0.0), axis=1, keepdims=True)
    r1 = jnp.sum(jnp.where(lanes == e1, tot0 + cum1 + po, 0.0), axis=1,
                 keepdims=True)
    pos_out[...] = jnp.concatenate([r0, r1], axis=1).astype(jnp.int32)
    w_out[...] = tkb[:, 2:4]
    w0_out[...] = jnp.broadcast_to(tkb[:, 2:3], (S, 128))
    w1_out[...] = jnp.broadcast_to(tkb[:, 3:4], (S, 128))
    po_out[...] = po.astype(jnp.int32)
    pc_out[...] = pc.astype(jnp.int32)


def _ksg_body(x2_hbm, posf_hbm, w16_hbm, xg_hbm, wout_hbm, posv, rowsv,
              wrowv, sem, *, S, PAIRS_W, CH):
    wid = lax.axis_index("s") * NC + lax.axis_index("c")
    half = PAIRS_W * (NW // 2)
    base = wid * PAIRS_W
    tok_base = jnp.where(base >= half, base - half, base)
    for ch in range(PAIRS_W // CH):
        b = base + ch * CH
        tb = tok_base + ch * CH
        pltpu.sync_copy(posf_hbm.at[pl.ds(b, CH)], posv)
        pltpu.sync_copy(x2_hbm.at[pl.ds(tb, CH)], rowsv)
        pltpu.async_copy(rowsv, xg_hbm.at[posv], sem).wait()
        pltpu.sync_copy(w16_hbm.at[pl.ds(b, CH)], wrowv)
        pltpu.async_copy(wrowv, wout_hbm.at[posv], sem).wait()


def _kc_body(pout_hbm, posf_hbm, pout2_hbm, posv, rowsv, sem, *,
             PAIRS_W, CH):
    wid = lax.axis_index("s") * NC + lax.axis_index("c")
    base = wid * PAIRS_W
    for ch in range(PAIRS_W // CH):
        b = base + ch * CH
        pltpu.sync_copy(posf_hbm.at[pl.ds(b, CH)], posv)
        pltpu.async_copy(pout_hbm.at[posv], rowsv, sem).wait()
        pltpu.sync_copy(rowsv, pout2_hbm.at[pl.ds(b, CH)])


def _ka_body(hid_ref, p0_ref, p1_ref, out_ref):
    out_ref[...] = hid_ref[...] + p0_ref[...] + p1_ref[...]


def _kf_body(po_ref, pc_ref, xg_ref, wv_ref, gw_ref, gb_ref, uw_ref, ub_ref,
             dw_ref, db_ref, out_ref, *, BM, E):
    g = pl.program_id(0)
    nb = (po_ref[E - 1] + pc_ref[E - 1]) // BM

    @pl.when(g < nb)
    def _():
        xb = xg_ref[...]
        gg = jnp.dot(xb, gw_ref[0], preferred_element_type=jnp.float32)
        gg = gg + gb_ref[0]
        u = jnp.dot(xb, uw_ref[0], preferred_element_type=jnp.float32)
        u = u + ub_ref[0]
        w0 = jnp.minimum(gg, 7.0)
        w1 = jnp.clip(u, -7.0, 7.0)
        glu = w0 * (1.0 / (1.0 + jnp.exp(-ALPHA * w0)))
        inter = (w1 + 1.0) * glu
        dn = jnp.dot(inter, dw_ref[0], preferred_element_type=jnp.float32)
        dn = dn + db_ref[0]
        out_ref[...] = dn * wv_ref[:, 0:1]


def _run_sc_scatter(x2, posf, w16, S, D, PAD):
    PAIRS_W = (2 * S) // NW
    CH = 64
    while PAIRS_W % CH:
        CH //= 2
    mesh = plsc.VectorSubcoreMesh(core_axis_name="c", subcore_axis_name="s")
    f = functools.partial(
        pl.kernel,
        functools.partial(_ksg_body, S=S, PAIRS_W=PAIRS_W, CH=CH),
        out_type=[jax.ShapeDtypeStruct((PAD, D), jnp.float32),
                  jax.ShapeDtypeStruct((PAD, 128), jnp.float32)],
        mesh=mesh,
        scratch_types=[pltpu.VMEM((CH,), jnp.int32),
                       pltpu.VMEM((CH, D), jnp.float32),
                       pltpu.VMEM((CH, 128), jnp.float32),
                       pltpu.SemaphoreType.DMA],
    )()
    return f(x2, posf, w16)


def _run_sc_combine(hid, pairout, posf, S, D):
    PAIRS_W = (2 * S) // NW
    CH = 64
    while PAIRS_W % CH:
        CH //= 2
    mesh = plsc.VectorSubcoreMesh(core_axis_name="c", subcore_axis_name="s")
    f = functools.partial(
        pl.kernel,
        functools.partial(_kc_body, PAIRS_W=PAIRS_W, CH=CH),
        out_type=jax.ShapeDtypeStruct((2 * S, D), jnp.float32),
        mesh=mesh,
        scratch_types=[pltpu.VMEM((CH,), jnp.int32),
                       pltpu.VMEM((CH, D), jnp.float32),
                       pltpu.SemaphoreType.DMA],
    )()
    pout2 = f(pairout, posf)
    BS = min(512, S)
    NI = S // BS
    return pl.pallas_call(
        _ka_body,
        grid=(NI,),
        in_specs=[
            pl.BlockSpec((BS, D), lambda i: (i, 0)),
            pl.BlockSpec((BS, D), lambda i: (i, 0)),
            pl.BlockSpec((BS, D), lambda i, NI=NI: (i + NI, 0)),
        ],
        out_specs=pl.BlockSpec((BS, D), lambda i: (i, 0)),
        out_shape=jax.ShapeDtypeStruct((S, D), jnp.float32),
    )(hid, pout2, pout2)


def kernel(hidden_states, ln1_w, q_w, q_b, k_w, k_b, v_w, v_b, o_w, o_b,
           sinks, ln2_w, router_w, router_b, gate_w, gate_b, up_w, up_b,
           down_w, down_b):
    B, S, D = hidden_states.shape
    H = sinks.shape[0]
    DQ = q_w.shape[1]
    DKV = k_w.shape[1]
    HD = DQ // H
    KV = DKV // HD
    E, _, F = gate_w.shape
    BM = 256
    PAD = 2 * S + E * BM
    G = PAD // BM

    x = hidden_states.reshape(S, D)
    f32 = jnp.float32
    bf16 = jnp.bfloat16

    BS1 = min(512, S)
    qkv_w = jnp.concatenate([q_w, k_w, v_w], axis=1)
    qkv_b = jnp.concatenate([q_b, k_b, v_b]).reshape(1, -1)
    cos, sin = _rope_tables(S, HD)
    cq = jnp.tile(cos, (1, H))
    sq = jnp.tile(sin, (1, H))
    ck = jnp.tile(cos, (1, KV))
    sk = jnp.tile(sin, (1, KV))

    q, k, v = pl.pallas_call(
        functools.partial(_k1_body, DQ=DQ, DKV=DKV, HD=HD),
        grid=(S // BS1,),
        in_specs=[
            pl.BlockSpec((BS1, D), lambda i: (i, 0)),
            pl.BlockSpec((1, D), lambda i: (0, 0)),
            pl.BlockSpec((D, DQ + 2 * DKV), lambda i: (0, 0)),
            pl.BlockSpec((1, DQ + 2 * DKV), lambda i: (0, 0)),
            pl.BlockSpec((BS1, DQ), lambda i: (i, 0)),
            pl.BlockSpec((BS1, DQ), lambda i: (i, 0)),
            pl.BlockSpec((BS1, DKV), lambda i: (i, 0)),
            pl.BlockSpec((BS1, DKV), lambda i: (i, 0)),
        ],
        out_specs=[
            pl.BlockSpec((BS1, DQ), lambda i: (i, 0)),
            pl.BlockSpec((BS1, DKV), lambda i: (i, 0)),
            pl.BlockSpec((BS1, DKV), lambda i: (i, 0)),
        ],
        out_shape=[
            jax.ShapeDtypeStruct((S, DQ), bf16),
            jax.ShapeDtypeStruct((S, DKV), bf16),
            jax.ShapeDtypeStruct((S, DKV), bf16),
        ],
    )(x, ln1_w.reshape(1, D), qkv_w, qkv_b, cq, sq, ck, sk)

    qh = q.reshape(S, H, HD).transpose(1, 0, 2)
    kh = k.reshape(S, KV, HD).transpose(1, 0, 2)
    vh = v.reshape(S, KV, HD).transpose(1, 0, 2)
    snk3 = sinks.reshape(H, 1, 1)
    BQ = min(512, S)
    rep = H // KV
    scale = 1.0 / float(np.sqrt(HD))
    parts = []
    for blk in range(S // BQ):
        L = (blk + 1) * BQ
        part = pl.pallas_call(
            functools.partial(_k2_body, BQ=BQ, L=L, row0=blk * BQ,
                              scale=scale),
            grid=(H,),
            in_specs=[
                pl.BlockSpec((1, BQ, HD), lambda h: (h, 0, 0)),
                pl.BlockSpec((1, L, HD), lambda h: (h // rep, 0, 0)),
                pl.BlockSpec((1, L, HD), lambda h: (h // rep, 0, 0)),
                pl.BlockSpec((1, 1, 1), lambda h: (h, 0, 0)),
            ],
            out_specs=pl.BlockSpec((1, BQ, HD), lambda h: (h, 0, 0)),
            out_shape=jax.ShapeDtypeStruct((H, BQ, HD), bf16),
        )(lax.slice_in_dim(qh, blk * BQ, L, axis=1),
          lax.slice_in_dim(kh, 0, L, axis=1),
          lax.slice_in_dim(vh, 0, L, axis=1), snk3)
        parts.append(part)
    attn = jnp.concatenate(parts, axis=1)

    att2 = attn.transpose(1, 0, 2).reshape(S, DQ)

    BS3 = min(512, S)
    hid, x2, tk = pl.pallas_call(
        functools.partial(_k3_body, E=E),
        grid=(S // BS3,),
        in_specs=[
            pl.BlockSpec((BS3, DQ), lambda i: (i, 0)),
            pl.BlockSpec((DQ, D), lambda i: (0, 0)),
            pl.BlockSpec((1, D), lambda i: (0, 0)),
            pl.BlockSpec((BS3, D), lambda i: (i, 0)),
            pl.BlockSpec((1, D), lambda i: (0, 0)),
            pl.BlockSpec((D, E), lambda i: (0, 0)),
            pl.BlockSpec((1, E), lambda i: (0, 0)),
        ],
        out_specs=[
            pl.BlockSpec((BS3, D), lambda i: (i, 0)),
            pl.BlockSpec((BS3, D), lambda i: (i, 0)),
            pl.BlockSpec((BS3, E), lambda i: (i, 0)),
        ],
        out_shape=[
            jax.ShapeDtypeStruct((S, D), f32),
            jax.ShapeDtypeStruct((S, D), f32),
            jax.ShapeDtypeStruct((S, E), f32),
        ],
    )(att2, o_w, o_b.reshape(1, D), x, ln2_w.reshape(1, D),
      router_w, router_b.reshape(1, E))

    Lnp = np.tril(np.ones((S, S), np.float32), -1).astype(jnp.bfloat16)
    Unp = np.triu(np.ones((E, E), np.float32), 1)
    pos, w2, w0b, w1b, po, pc = pl.pallas_call(
        functools.partial(_kr_body, S=S, E=E, BM=BM),
        grid=(1,),
        in_specs=[
            pl.BlockSpec((S, E), lambda i: (0, 0)),
            pl.BlockSpec((S, S), lambda i: (0, 0)),
            pl.BlockSpec((E, E), lambda i: (0, 0)),
        ],
        out_specs=[
            pl.BlockSpec((S, 2), lambda i: (0, 0)),
            pl.BlockSpec((S, 2), lambda i: (0, 0)),
            pl.BlockSpec((S, 128), lambda i: (0, 0)),
            pl.BlockSpec((S, 128), lambda i: (0, 0)),
            pl.BlockSpec((1, E), lambda i: (0, 0)),
            pl.BlockSpec((1, E), lambda i: (0, 0)),
        ],
        out_shape=[
            jax.ShapeDtypeStruct((S, 2), jnp.int32),
            jax.ShapeDtypeStruct((S, 2), f32),
            jax.ShapeDtypeStruct((S, 128), f32),
            jax.ShapeDtypeStruct((S, 128), f32),
            jax.ShapeDtypeStruct((1, E), jnp.int32),
            jax.ShapeDtypeStruct((1, E), jnp.int32),
        ],
    )(tk, Lnp, Unp)

    posf = pos.T.reshape(2 * S)

    w16 = jnp.concatenate([w0b, w1b], axis=0)
    xg, wv16 = _run_sc_scatter(x2, posf, w16, S, D, PAD)

    grid_spec = pltpu.PrefetchScalarGridSpec(
        num_scalar_prefetch=2,
        grid=(G,),
        in_specs=[
            pl.BlockSpec((BM, D), lambda g, po_r, pc_r: (g, 0)),
            pl.BlockSpec((BM, 128), lambda g, po_r, pc_r: (g, 0)),
            pl.BlockSpec((1, D, F),
                         lambda g, po_r, pc_r: (_e_of(g, po_r, BM, E), 0, 0)),
            pl.BlockSpec((1, 1, F),
                         lambda g, po_r, pc_r: (_e_of(g, po_r, BM, E), 0, 0)),
            pl.BlockSpec((1, D, F),
                         lambda g, po_r, pc_r: (_e_of(g, po_r, BM, E), 0, 0)),
            pl.BlockSpec((1, 1, F),
                         lambda g, po_r, pc_r: (_e_of(g, po_r, BM, E), 0, 0)),
            pl.BlockSpec((1, F, D),
                         lambda g, po_r, pc_r: (_e_of(g, po_r, BM, E), 0, 0)),
            pl.BlockSpec((1, 1, D),
                         lambda g, po_r, pc_r: (_e_of(g, po_r, BM, E), 0, 0)),
        ],
        out_specs=pl.BlockSpec((BM, D), lambda g, po_r, pc_r: (g, 0)),
    )
    pairout = pl.pallas_call(
        functools.partial(_kf_body, BM=BM, E=E),
        grid_spec=grid_spec,
        out_shape=jax.ShapeDtypeStruct((PAD, D), f32),
        compiler_params=pltpu.CompilerParams(
            vmem_limit_bytes=56 * 1024 * 1024),
    )(po.reshape(E), pc.reshape(E), xg, wv16,
      gate_w, gate_b.reshape(E, 1, F),
      up_w, up_b.reshape(E, 1, F),
      down_w, down_b.reshape(E, 1, D))

    out = _run_sc_combine(hid, pairout, posf, S, D)
    return out.reshape(B, S, D)


def _e_of(g, po_ref, BM, E):
    acc = jnp.int32(-1)
    for ee in range(E):
        acc = acc + jnp.where(po_ref[ee] <= g * BM, 1, 0).astype(jnp.int32)
    return acc
